# baseline (speedup 1.0000x reference)
import functools

import jax
import jax.numpy as jnp
from jax import lax
from jax.experimental import pallas as pl
from jax.experimental.pallas import tpu as pltpu
from jax.experimental.pallas import tpu_sc as plsc

VOCAB = 8192
BATCH = 4096
D = 8192

_info = plsc.get_sparse_core_info()
NC, NS = _info.num_cores, _info.num_subcores
NW = NC * NS
B_PER_W = BATCH // NW
CHUNK = 2
NBUF = 4
SBUF = 3
NCHUNK = B_PER_W // CHUNK

_mesh = plsc.VectorSubcoreMesh(core_axis_name="c", subcore_axis_name="s")


@functools.partial(
    pl.kernel,
    mesh=_mesh,
    out_type=jax.ShapeDtypeStruct((BATCH, D), jnp.float32),
    scratch_types=[
        pltpu.VMEM((NCHUNK, CHUNK), jnp.int32),
        [pltpu.VMEM((CHUNK, D), jnp.float32) for _ in range(NBUF)],
        pltpu.VMEM_SHARED((NS, SBUF, CHUNK, D), jnp.float32),
        [pltpu.SemaphoreType.DMA for _ in range(NBUF)],
        [pltpu.SemaphoreType.DMA for _ in range(SBUF)],
        [pltpu.SemaphoreType.DMA for _ in range(SBUF)],
        pltpu.SemaphoreType.DMA,
    ],
)
def _gather_rows(table_hbm, idx_hbm, out_hbm, idx_v, bufs, shared,
                 gsems, xsems, wsems, osem):
    cid = lax.axis_index("c")
    sid = lax.axis_index("s")
    wid = sid * NC + cid
    base = wid * B_PER_W
    pltpu.sync_copy(idx_hbm.at[wid], idx_v)

    def out_slice(j):
        return out_hbm.at[pl.ds(base + j * CHUNK, CHUNK)]

    for b in range(NBUF):
        pltpu.async_copy(table_hbm.at[idx_v.at[b]], bufs[b], gsems[b])

    def body(i, carry):
        for b in range(SBUF):
            k = NBUF * i + b

            @pl.when(k >= NBUF)
            def _():
                pltpu.make_async_copy(shared.at[sid, b],
                                      out_slice(k - NBUF), wsems[b]).wait()

            pltpu.make_async_copy(table_hbm.at[idx_v.at[k]], bufs[b],
                                  gsems[b]).wait()
            pltpu.async_copy(bufs[b], shared.at[sid, b], xsems[b])
            pltpu.make_async_copy(bufs[b], shared.at[sid, b],
                                  xsems[b]).wait()
            pltpu.async_copy(shared.at[sid, b], out_slice(k), wsems[b])

            @pl.when(k + NBUF < NCHUNK)
            def _():
                pltpu.async_copy(table_hbm.at[idx_v.at[k + NBUF]],
                                 bufs[b], gsems[b])

        k = NBUF * i + NBUF - 1
        db = NBUF - 1
        pltpu.make_async_copy(table_hbm.at[idx_v.at[k]], bufs[db],
                              gsems[db]).wait()
        pltpu.async_copy(bufs[db], out_slice(k), osem)

        @pl.when(k + NBUF < NCHUNK)
        def _():
            pltpu.make_async_copy(bufs[db], out_slice(k), osem).wait()
            pltpu.async_copy(table_hbm.at[idx_v.at[k + NBUF]], bufs[db],
                             gsems[db])

        return carry

    lax.fori_loop(0, NCHUNK // NBUF, body, 0, unroll=False)

    for b in range(SBUF):
        j = NCHUNK - NBUF + b
        pltpu.make_async_copy(shared.at[sid, b], out_slice(j),
                              wsems[b]).wait()
    pltpu.make_async_copy(bufs[NBUF - 1], out_slice(NCHUNK - 1), osem).wait()


def kernel(x, table):
    idx = x.reshape(NW, NCHUNK, CHUNK).astype(jnp.int32)
    return _gather_rows(table, idx)
